# baseline (speedup 1.0000x reference)
import functools

import jax
import jax.numpy as jnp
from jax import lax
from jax.experimental import pallas as pl
from jax.experimental.pallas import tpu as pltpu
from jax.experimental.pallas import tpu_sc as plsc

_INFO = plsc.get_sparse_core_info()
_NC = _INFO.num_cores
_NS = _INFO.num_subcores
_NW = _NC * _NS

_NB = 16


@functools.lru_cache(maxsize=None)
def _make_tgather(batch: int, hist: int, dim: int):
    b_per_w = batch // _NW
    n_sub = b_per_w // _NB
    n_chunks = batch // _NB
    assert batch % (_NW * _NB) == 0 and n_sub >= 4 and n_sub % 2 == 0
    mesh = plsc.VectorSubcoreMesh(core_axis_name="c", subcore_axis_name="s")

    @functools.partial(
        pl.kernel,
        mesh=mesh,
        out_type=jax.ShapeDtypeStruct((hist * dim, n_chunks, _NB), jnp.float32),
        scratch_types=[
            *[pltpu.VMEM((hist, _NB), jnp.int32) for _ in range(2)],
            *[pltpu.VMEM((hist * _NB,), jnp.int32) for _ in range(2)],
            *[pltpu.VMEM((hist * _NB, dim), jnp.float32) for _ in range(2)],
            *[pltpu.VMEM((hist * dim, _NB), jnp.float32) for _ in range(2)],
            *[pltpu.SemaphoreType.DMA for _ in range(6)],
        ],
        compiler_params=pltpu.CompilerParams(
            use_tc_tiling_on_sc=False, needs_layout_passes=False),
    )
    def tgather_kernel(idxT_hbm, table_hbm, out_hbm, iv0, iv1, if0, if1,
                       g0, g1, s0, s1, *sems):
        iv = (iv0, iv1)
        ifl = (if0, if1)
        G = (g0, g1)
        S = (s0, s1)
        si = sems[0:2]
        sg = sems[2:4]
        so = sems[4:6]
        wid = lax.axis_index("s") * _NC + lax.axis_index("c")
        base_b = wid * b_per_w
        base_m = wid * n_sub
        lane = lax.iota(jnp.int32, _NB)

        def idx_copy(m, p):
            pltpu.async_copy(
                idxT_hbm.at[:, pl.ds(base_b + m * _NB, _NB)], iv[p], si[p])

        def wait_idx(p):
            pltpu.make_async_copy(
                idxT_hbm.at[:, pl.ds(base_b, _NB)], iv[p], si[p]).wait()

        def repack_idx(p):
            @pl.loop(0, hist)
            def _h(h):
                ifl[p][pl.ds(h * _NB, _NB)] = iv[p][h, :]

        def gather(p):
            pltpu.async_copy(table_hbm.at[ifl[p]], G[p], sg[p])

        def wait_gather(p):
            pltpu.make_async_copy(table_hbm.at[ifl[p]], G[p], sg[p]).wait()

        def out_copy(m, p):
            pltpu.async_copy(
                S[p], out_hbm.at[:, base_m + m, :], so[p])

        def wait_out(p):
            pltpu.make_async_copy(
                S[p], out_hbm.at[:, base_m, :], so[p]).wait()

        def transpose(p):
            @pl.loop(0, hist // 2)
            def _h(hh):
                h0 = 2 * hh
                rows = [h0 * _NB + lane, (h0 + 1) * _NB + lane]
                sbases = [h0 * dim, (h0 + 1) * dim]
                for c in range(dim):
                    cv = jnp.full((_NB,), c, jnp.int32)
                    for u in range(2):
                        vals = plsc.load_gather(G[p], [rows[u], cv])
                        S[p][sbases[u] + c, :] = vals

        def body(m, p, do_next_gather, do_idx_prefetch, do_wait_out):
            wait_gather(p)
            if do_next_gather:
                wait_idx(1 - p)
                repack_idx(1 - p)
                gather(1 - p)
            if do_idx_prefetch:
                idx_copy(m + 2, p)
            if do_wait_out:
                wait_out(p)
            transpose(p)
            out_copy(m, p)

        idx_copy(0, 0)
        wait_idx(0)
        repack_idx(0)
        gather(0)
        idx_copy(1, 1)
        body(0, 0, True, True, False)
        body(1, 1, True, True, False)

        @pl.loop(0, (n_sub - 4) // 2)
        def _steady(t):
            m = 2 + 2 * t
            body(m, 0, True, True, True)
            body(m + 1, 1, True, True, True)

        body(n_sub - 2, 0, True, False, True)
        body(n_sub - 1, 1, False, False, True)
        wait_out(0)
        wait_out(1)

    return tgather_kernel


def kernel(indices, table):
    batch, hist = indices.shape
    dim = table.shape[1]
    idx_t = jnp.transpose(indices)
    out_t = _make_tgather(batch, hist, dim)(idx_t, table)
    out_t = out_t.reshape(hist, dim, batch)
    return jnp.transpose(out_t, (2, 0, 1))


# speedup vs baseline: 1.0012x; 1.0012x over previous
import functools

import jax
import jax.numpy as jnp
from jax import lax
from jax.experimental import pallas as pl
from jax.experimental.pallas import tpu as pltpu
from jax.experimental.pallas import tpu_sc as plsc

_INFO = plsc.get_sparse_core_info()
_NC = _INFO.num_cores
_NS = _INFO.num_subcores
_NW = _NC * _NS

_NB = 16


@functools.lru_cache(maxsize=None)
def _make_tgather(batch: int, hist: int, dim: int):
    b_per_w = batch // _NW
    n_sub = b_per_w // _NB
    n_chunks = batch // _NB
    assert batch % (_NW * _NB) == 0 and n_sub >= 4 and n_sub % 2 == 0
    assert hist % 2 == 0 and dim % _NB == 0
    mesh = plsc.VectorSubcoreMesh(core_axis_name="c", subcore_axis_name="s")

    @functools.partial(
        pl.kernel,
        mesh=mesh,
        out_type=jax.ShapeDtypeStruct((hist * dim, n_chunks, _NB), jnp.float32),
        scratch_types=[
            *[pltpu.VMEM((hist, _NB), jnp.int32) for _ in range(2)],
            *[pltpu.VMEM((hist * _NB,), jnp.int32) for _ in range(2)],
            *[pltpu.VMEM((hist * _NB, dim), jnp.float32) for _ in range(2)],
            *[pltpu.VMEM((hist * dim, _NB), jnp.float32) for _ in range(2)],
            *[pltpu.SemaphoreType.DMA for _ in range(6)],
        ],
        compiler_params=pltpu.CompilerParams(
            use_tc_tiling_on_sc=False, needs_layout_passes=False),
    )
    def tgather_kernel(idxT_hbm, table_hbm, out_hbm, iv0, iv1, if0, if1,
                       g0, g1, s0, s1, *sems):
        iv = (iv0, iv1)
        ifl = (if0, if1)
        G = (g0, g1)
        S = (s0, s1)
        si = sems[0:2]
        sg = sems[2:4]
        so = sems[4:6]
        wid = lax.axis_index("s") * _NC + lax.axis_index("c")
        base_b = wid * b_per_w
        base_m = wid * n_sub
        lane = lax.iota(jnp.int32, _NB)

        def idx_copy(m, p):
            pltpu.async_copy(
                idxT_hbm.at[:, pl.ds(base_b + m * _NB, _NB)], iv[p], si[p])

        def wait_idx(p):
            pltpu.make_async_copy(
                idxT_hbm.at[:, pl.ds(base_b, _NB)], iv[p], si[p]).wait()

        def repack_idx(p):
            @pl.loop(0, hist)
            def _h(h):
                ifl[p][pl.ds(h * _NB, _NB)] = iv[p][h, :]

        def gather(p):
            pltpu.async_copy(table_hbm.at[ifl[p]], G[p], sg[p])

        def wait_gather(p):
            pltpu.make_async_copy(table_hbm.at[ifl[p]], G[p], sg[p]).wait()

        def out_copy(m, p):
            pltpu.async_copy(
                S[p], out_hbm.at[:, base_m + m, :], so[p])

        def wait_out(p):
            pltpu.make_async_copy(
                S[p], out_hbm.at[:, base_m, :], so[p]).wait()

        def transpose(p):
            @pl.loop(0, hist // 2)
            def _h(hh):
                h0 = 2 * hh
                rows = [h0 * _NB + lane, (h0 + 1) * _NB + lane]
                sbases = [h0 * dim, (h0 + 1) * dim]
                for c in range(dim):
                    cv = jnp.full((_NB,), c, jnp.int32)
                    for u in range(2):
                        vals = plsc.load_gather(G[p], [rows[u], cv])
                        S[p][sbases[u] + c, :] = vals

        def body(m, p, do_next_gather, do_idx_prefetch, do_wait_out):
            wait_gather(p)
            if do_next_gather:
                wait_idx(1 - p)
                repack_idx(1 - p)
                gather(1 - p)
            if do_idx_prefetch:
                idx_copy(m + 2, p)
            if do_wait_out:
                wait_out(p)
            transpose(p)
            out_copy(m, p)

        idx_copy(0, 0)
        wait_idx(0)
        repack_idx(0)
        gather(0)
        idx_copy(1, 1)
        body(0, 0, True, True, False)
        body(1, 1, True, True, False)

        @pl.loop(0, (n_sub - 4) // 2)
        def _steady(t):
            m = 2 + 2 * t
            body(m, 0, True, True, True)
            body(m + 1, 1, True, True, True)

        body(n_sub - 2, 0, True, False, True)
        body(n_sub - 1, 1, False, False, True)
        wait_out(0)
        wait_out(1)

    return tgather_kernel


def kernel(indices, table):
    batch, hist = indices.shape
    dim = table.shape[1]
    idx_t = jnp.transpose(indices)
    out_t = _make_tgather(batch, hist, dim)(idx_t, table)
    out_t = out_t.reshape(hist, dim, batch)
    return jnp.transpose(out_t, (2, 0, 1))
